# baseline (speedup 1.0000x reference)
import functools

import jax
import jax.numpy as jnp
from jax import lax
from jax.experimental import pallas as pl
from jax.experimental.pallas import tpu as pltpu
from jax.experimental.pallas import tpu_sc as plsc

N = 50000
E = 800000
H = 64
DZ = 32
DIN = 6
DOUT = 3

NC = 2
NS = 16
CHUNK = 128
E_PAD = 802816
EPT = E_PAD // NS
NCH_T = EPT // CHUNK
EPW = E_PAD // (NC * NS)
NCH_W = EPW // CHUNK
ROWS_ACC = 50176
RPT = ROWS_ACC // NS

_MESH = plsc.VectorSubcoreMesh(core_axis_name="c", subcore_axis_name="s")
_SC_PARAMS = pltpu.CompilerParams(use_tc_tiling_on_sc=False)


def _make_segsum(specs, sup, out_groups):
    batch = sup * CHUNK

    @functools.partial(
        pl.kernel,
        out_type=jax.ShapeDtypeStruct((out_groups * ROWS_ACC, 16), jnp.float32),
        mesh=_MESH,
        scratch_types=[
            pltpu.VMEM((2, batch), jnp.int32),
            pltpu.VMEM((2, sup, CHUNK), jnp.int32),
            pltpu.VMEM((2, batch, 16), jnp.float32),
            pltpu.VMEM_SHARED((ROWS_ACC, 16), jnp.float32),
            pltpu.SemaphoreType.DMA,
            pltpu.SemaphoreType.DMA,
            pltpu.SemaphoreType.DMA,
        ],
        compiler_params=_SC_PARAMS,
    )
    def k(tabh, srch, dsth2, zch, out, sidx, didx, rows, acc,
          sem_i, sem_g, sem_s):
        c = lax.axis_index("c")
        s = lax.axis_index("s")

        for g, (src_base_fn, out_base_fn, iters, supg) in enumerate(specs):
            batchg = supg * CHUNK
            sbase = src_base_fn(c, s)
            drow = sbase % E_PAD // CHUNK

            pltpu.sync_copy(zch, acc.at[pl.ds(s * RPT, RPT)])
            plsc.subcore_barrier()

            def fire_idx(i, b):
                pltpu.async_copy(srch.at[pl.ds(sbase + i * batchg, batchg)],
                                 sidx.at[b, pl.ds(0, batchg)], sem_i)
                pltpu.async_copy(dsth2.at[pl.ds(drow + i * supg, supg)],
                                 didx.at[b, pl.ds(0, supg)], sem_i)

            def wait_idx(b):
                pltpu.make_async_copy(srch.at[pl.ds(sbase, batchg)],
                                      sidx.at[b, pl.ds(0, batchg)], sem_i).wait()
                pltpu.make_async_copy(dsth2.at[pl.ds(drow, supg)],
                                      didx.at[b, pl.ds(0, supg)], sem_i).wait()

            def drain_scatters(b):
                for j in range(supg):
                    pltpu.make_async_copy(
                        rows.at[b, pl.ds(j * CHUNK, CHUNK)],
                        acc.at[didx.at[b, j]], sem_s).wait()

            fire_idx(0, 0)

            def body(i, carry):
                b = i % 2

                wait_idx(b)
                for j in range(supg):
                    pltpu.async_copy(
                        tabh.at[sidx.at[b, pl.ds(j * CHUNK, CHUNK)]],
                        rows.at[b, pl.ds(j * CHUNK, CHUNK)], sem_g)

                @pl.when(i > 0)
                def _():
                    drain_scatters(1 - b)

                @pl.when(i < iters - 1)
                def _():
                    fire_idx(i + 1, 1 - b)

                for j in range(supg):
                    pltpu.make_async_copy(
                        tabh.at[sidx.at[b, pl.ds(j * CHUNK, CHUNK)]],
                        rows.at[b, pl.ds(j * CHUNK, CHUNK)], sem_g).wait()
                for j in range(supg):
                    pltpu.async_copy(rows.at[b, pl.ds(j * CHUNK, CHUNK)],
                                     acc.at[didx.at[b, j]], sem_s, add=True)
                return carry

            lax.fori_loop(0, iters, body, 0)
            drain_scatters((iters - 1) % 2)
            plsc.subcore_barrier()
            pltpu.sync_copy(acc.at[pl.ds(s * RPT, RPT)],
                            out.at[pl.ds(out_base_fn(c) + s * RPT, RPT)])
            plsc.subcore_barrier()

    return k


_SUP = 14
_ITERS_C = EPT // (_SUP * CHUNK)
_SUP_R = 14
_ITERS_R = EPW // (_SUP_R * CHUNK)

_segsum_half_k = [_make_segsum(
    [(lambda c, s, h=h: (2 * h + c) * E_PAD + s * EPT,
      lambda c: c * ROWS_ACC, _ITERS_C, _SUP)],
    _SUP, 2) for h in range(2)]

_segsum_rows16_k = _make_segsum(
    [(lambda c, s: (s * NC + c) * EPW,
      lambda c: c * ROWS_ACC, _ITERS_R, _SUP_R)],
    _SUP, 2)


def _segsum_cols(yq, src4, dstp, zc16):
    d2 = dstp.reshape(E_PAD // CHUNK, CHUNK)
    out = jnp.concatenate([_segsum_half_k[0](yq, src4, d2, zc16),
                           _segsum_half_k[1](yq, src4, d2, zc16)], axis=0)
    return out.reshape(4, ROWS_ACC, 16)


def _segsum_rows16(u, srcp, dstp, zc16):
    out = _segsum_rows16_k(u, srcp, dstp.reshape(E_PAD // CHUNK, CHUNK), zc16)
    return out.reshape(2, ROWS_ACC, 16)


BN = 2000
GRID = N // BN

def _dot(a, b):
    return jnp.dot(a, b, preferred_element_type=jnp.float32)


def _row_spec(w):
    return pl.BlockSpec((BN, w), lambda i: (i, 0))


def _stack_spec(n, w):
    return pl.BlockSpec((n, BN, w), lambda i: (0, i, 0))


def _full_spec(shape):
    return pl.BlockSpec(shape, lambda i: tuple(0 for _ in shape))


def _tc_first(inp16, aggI2, Ws1p, Wn1p, b1):
    def body(inp_r, a_r, Ws1_r, Wn1_r, b1_r, f_o, ai_o):
        a = a_r[...]
        aggI = a[0] + a[1]
        feats = jnp.maximum(_dot(inp_r[...], Ws1_r[...]) + _dot(aggI, Wn1_r[...])
                            + b1_r[...], 0.0)
        for q in range(4):
            f_o[q] = feats[:, 16 * q:16 * (q + 1)]
        ai_o[...] = aggI

    return pl.pallas_call(
        body,
        grid=(GRID,),
        in_specs=[_row_spec(16), _stack_spec(2, 16),
                  _full_spec((16, 64)), _full_spec((16, 64)), _full_spec((1, 64))],
        out_specs=[_stack_spec(4, 16), _row_spec(16)],
        out_shape=[jax.ShapeDtypeStruct((4, N, 16), jnp.float32),
                   jax.ShapeDtypeStruct((N, 16), jnp.float32)],
    )(inp16, aggI2, Ws1p, Wn1p, b1)


def _tc_block(fquad, aggF4, aggZ2, aggI, inp16,
              Ws64, Ws32, Ws16, Wn64, Wn32, Wn16, b, z):
    def body(f_r, af_r, az_r, ai_r, inp_r,
             Ws64_r, Ws32_r, Ws16_r, Wn64_r, Wn32_r, Wn16_r, b_r, z_r,
             f_o):
        f = f_r[...]
        feats = jnp.concatenate([f[0], f[1], f[2], f[3]], axis=1)
        af = af_r[...]
        aggF = jnp.concatenate([af[0], af[1], af[2], af[3]], axis=1)
        az = az_r[...]
        aggZ = jnp.concatenate([az[0], az[1]], axis=1)
        pre = (_dot(feats, Ws64_r[...]) + _dot(z_r[...], Ws32_r[...])
               + _dot(inp_r[...], Ws16_r[...])
               + _dot(aggF, Wn64_r[...]) + _dot(aggZ, Wn32_r[...])
               + _dot(ai_r[...], Wn16_r[...]) + b_r[...])
        feats = jnp.maximum(pre, 0.0)
        for q in range(4):
            f_o[q] = feats[:, 16 * q:16 * (q + 1)]

    return pl.pallas_call(
        body,
        grid=(GRID,),
        in_specs=[_stack_spec(4, 16), _stack_spec(4, 16), _stack_spec(2, 16),
                  _row_spec(16), _row_spec(16),
                  _full_spec((64, 64)), _full_spec((32, 64)), _full_spec((16, 64)),
                  _full_spec((64, 64)), _full_spec((32, 64)), _full_spec((16, 64)),
                  _full_spec((1, 64)), _row_spec(32)],
        out_specs=_stack_spec(4, 16),
        out_shape=jax.ShapeDtypeStruct((4, N, 16), jnp.float32),
    )(fquad, aggF4, aggZ2, aggI, inp16, Ws64, Ws32, Ws16, Wn64, Wn32, Wn16, b, z)


def _tc_last(fquad, aggF4, Wsl16, Wnl16, bl16):
    def body(f_r, af_r, Wsl_r, Wnl_r, bl_r, o):
        f = f_r[...]
        feats = jnp.concatenate([f[0], f[1], f[2], f[3]], axis=1)
        af = af_r[...]
        aggF = jnp.concatenate([af[0], af[1], af[2], af[3]], axis=1)
        o[...] = _dot(feats, Wsl_r[...]) + _dot(aggF, Wnl_r[...]) + bl_r[...]

    return pl.pallas_call(
        body,
        grid=(GRID,),
        in_specs=[_stack_spec(4, 16), _stack_spec(4, 16),
                  _full_spec((64, 16)), _full_spec((64, 16)), _full_spec((1, 16))],
        out_specs=_row_spec(16),
        out_shape=jax.ShapeDtypeStruct((N, 16), jnp.float32),
    )(fquad, aggF4, Wsl16, Wnl16, bl16)


def kernel(vertices, edges, vector_potential, z, params):
    f32 = jnp.float32
    src = edges[:, 0]
    dst = edges[:, 1]
    pad = E_PAD - E
    srcp = jnp.concatenate([src, jnp.zeros((pad,), jnp.int32)])
    dstp = jnp.concatenate([dst, jnp.full((pad,), N, jnp.int32)])
    src4 = jnp.concatenate([srcp, srcp + N, srcp + 2 * N, srcp + 3 * N])

    inp16 = jnp.pad(jnp.concatenate([vertices, vector_potential], axis=-1),
                    ((0, 0), (0, 16 - 2 * 3)))
    zc16 = jnp.zeros((RPT, 16), f32)

    Ws1, Wn1, b1 = params["first"]
    Ws1p = jnp.pad(Ws1, ((0, 16 - DIN), (0, 0)))
    Wn1p = jnp.pad(Wn1, ((0, 16 - DIN), (0, 0)))
    b1r = b1.reshape(1, H)

    blocks = []
    for (Ws, Wn, b) in params["blocks"]:
        blocks.append((Ws[:H], Ws[H:H + DZ], jnp.pad(Ws[H + DZ:], ((0, 16 - DIN), (0, 0))),
                       Wn[:H], Wn[H:H + DZ], jnp.pad(Wn[H + DZ:], ((0, 16 - DIN), (0, 0))),
                       b.reshape(1, H)))

    Wsl, Wnl, bl = params["last"]
    Wsl16 = jnp.pad(Wsl, ((0, 0), (0, 16 - DOUT)))
    Wnl16 = jnp.pad(Wnl, ((0, 0), (0, 16 - DOUT)))
    bl16 = jnp.pad(bl, (0, 16 - DOUT)).reshape(1, 16)

    aggI2 = _segsum_rows16(inp16, srcp, dstp, zc16)
    zq01 = jnp.concatenate([z[0, :, :16], z[0, :, 16:],
                            z[1, :, :16], z[1, :, 16:]], axis=0)
    zq23 = jnp.concatenate([z[2, :, :16], z[2, :, 16:],
                            z[3, :, :16], z[3, :, 16:]], axis=0)
    aggZ01 = _segsum_cols(zq01, src4, dstp, zc16)
    aggZ23 = _segsum_cols(zq23, src4, dstp, zc16)
    aggZ = [aggZ01[0:2], aggZ01[2:4], aggZ23[0:2], aggZ23[2:4]]

    fquad, aggI = _tc_first(inp16, aggI2, Ws1p, Wn1p, b1r)

    for i in range(4):
        aggF4 = _segsum_cols(fquad.reshape(4 * N, 16), src4, dstp, zc16)
        fquad = _tc_block(fquad, aggF4, aggZ[i], aggI,
                          inp16, *blocks[i], z[i])

    aggF4 = _segsum_cols(fquad.reshape(4 * N, 16), src4, dstp, zc16)
    out16 = _tc_last(fquad, aggF4, Wsl16, Wnl16, bl16)
    return out16[:, :DOUT]


# speedup vs baseline: 1.0837x; 1.0837x over previous
import functools

import jax
import jax.numpy as jnp
from jax import lax
from jax.experimental import pallas as pl
from jax.experimental.pallas import tpu as pltpu
from jax.experimental.pallas import tpu_sc as plsc

N = 50000
E = 800000
H = 64
DZ = 32
DIN = 6
DOUT = 3

NC = 2
NS = 16
CHUNK = 128
E_PAD = 802816
EPT = E_PAD // NS
NCH_T = EPT // CHUNK
EPW = E_PAD // (NC * NS)
NCH_W = EPW // CHUNK
ROWS_ACC = 50176
RPT = ROWS_ACC // NS

_MESH = plsc.VectorSubcoreMesh(core_axis_name="c", subcore_axis_name="s")
_SC_PARAMS = pltpu.CompilerParams(use_tc_tiling_on_sc=False)


def _make_segsum(specs, sup, out_groups):
    batch = sup * CHUNK

    @functools.partial(
        pl.kernel,
        out_type=jax.ShapeDtypeStruct((out_groups * ROWS_ACC, 16), jnp.float32),
        mesh=_MESH,
        scratch_types=[
            pltpu.VMEM((2, batch), jnp.int32),
            pltpu.VMEM((2, sup, CHUNK), jnp.int32),
            pltpu.VMEM((2, batch, 16), jnp.float32),
            pltpu.VMEM_SHARED((ROWS_ACC, 16), jnp.float32),
            pltpu.SemaphoreType.DMA,
            pltpu.SemaphoreType.DMA,
            pltpu.SemaphoreType.DMA,
        ],
        compiler_params=_SC_PARAMS,
    )
    def k(tabh, srch, dsth2, zch, out, sidx, didx, rows, acc,
          sem_i, sem_g, sem_s):
        c = lax.axis_index("c")
        s = lax.axis_index("s")

        for g, (src_base_fn, out_base_fn, iters, supg) in enumerate(specs):
            batchg = supg * CHUNK
            sbase = src_base_fn(c, s)
            drow = sbase % E_PAD // CHUNK

            pltpu.sync_copy(zch, acc.at[pl.ds(s * RPT, RPT)])
            plsc.subcore_barrier()

            def fire_idx(i, b):
                pltpu.async_copy(srch.at[pl.ds(sbase + i * batchg, batchg)],
                                 sidx.at[b, pl.ds(0, batchg)], sem_i)
                pltpu.async_copy(dsth2.at[pl.ds(drow + i * supg, supg)],
                                 didx.at[b, pl.ds(0, supg)], sem_i)

            def wait_idx(b):
                pltpu.make_async_copy(srch.at[pl.ds(sbase, batchg)],
                                      sidx.at[b, pl.ds(0, batchg)], sem_i).wait()
                pltpu.make_async_copy(dsth2.at[pl.ds(drow, supg)],
                                      didx.at[b, pl.ds(0, supg)], sem_i).wait()

            def drain_scatters(b):
                for j in range(supg):
                    pltpu.make_async_copy(
                        rows.at[b, pl.ds(j * CHUNK, CHUNK)],
                        acc.at[didx.at[b, j]], sem_s).wait()

            fire_idx(0, 0)

            def body(i, carry):
                b = i % 2

                wait_idx(b)
                for j in range(supg):
                    pltpu.async_copy(
                        tabh.at[sidx.at[b, pl.ds(j * CHUNK, CHUNK)]],
                        rows.at[b, pl.ds(j * CHUNK, CHUNK)], sem_g)

                @pl.when(i > 0)
                def _():
                    drain_scatters(1 - b)

                @pl.when(i < iters - 1)
                def _():
                    fire_idx(i + 1, 1 - b)

                for j in range(supg):
                    pltpu.make_async_copy(
                        tabh.at[sidx.at[b, pl.ds(j * CHUNK, CHUNK)]],
                        rows.at[b, pl.ds(j * CHUNK, CHUNK)], sem_g).wait()
                for j in range(supg):
                    pltpu.async_copy(rows.at[b, pl.ds(j * CHUNK, CHUNK)],
                                     acc.at[didx.at[b, j]], sem_s, add=True)
                return carry

            lax.fori_loop(0, iters, body, 0)
            drain_scatters((iters - 1) % 2)
            plsc.subcore_barrier()
            pltpu.sync_copy(acc.at[pl.ds(s * RPT, RPT)],
                            out.at[pl.ds(out_base_fn(c) + s * RPT, RPT)])
            plsc.subcore_barrier()

    return k


_SUP = 14
_ITERS_C = EPT // (_SUP * CHUNK)
_SUP_R = 14
_ITERS_R = EPW // (_SUP_R * CHUNK)

_segsum_cols_k = _make_segsum(
    [(lambda c, s, g=g: (c * 2 + g) * E_PAD + s * EPT,
      lambda c, g=g: (c * 2 + g) * ROWS_ACC, _ITERS_C, _SUP)
     for g in range(2)],
    _SUP, 4)

_segsum_rows16_k = _make_segsum(
    [(lambda c, s: (s * NC + c) * EPW,
      lambda c: c * ROWS_ACC, _ITERS_R, _SUP_R)],
    _SUP, 2)


def _segsum_cols(yq, src4, dstp, zc16):
    out = _segsum_cols_k(yq, src4, dstp.reshape(E_PAD // CHUNK, CHUNK), zc16)
    return out.reshape(4, ROWS_ACC, 16)


def _segsum_rows16(u, srcp, dstp, zc16):
    out = _segsum_rows16_k(u, srcp, dstp.reshape(E_PAD // CHUNK, CHUNK), zc16)
    return out.reshape(2, ROWS_ACC, 16)


BN = 2000
GRID = N // BN

def _dot(a, b):
    return jnp.dot(a, b, preferred_element_type=jnp.float32)


def _row_spec(w):
    return pl.BlockSpec((BN, w), lambda i: (i, 0))


def _stack_spec(n, w):
    return pl.BlockSpec((n, BN, w), lambda i: (0, i, 0))


def _full_spec(shape):
    return pl.BlockSpec(shape, lambda i: tuple(0 for _ in shape))


def _tc_first(inp16, aggI2, Ws1p, Wn1p, b1):
    def body(inp_r, a_r, Ws1_r, Wn1_r, b1_r, f_o, ai_o):
        a = a_r[...]
        aggI = a[0] + a[1]
        feats = jnp.maximum(_dot(inp_r[...], Ws1_r[...]) + _dot(aggI, Wn1_r[...])
                            + b1_r[...], 0.0)
        for q in range(4):
            f_o[q] = feats[:, 16 * q:16 * (q + 1)]
        ai_o[...] = aggI

    return pl.pallas_call(
        body,
        grid=(GRID,),
        in_specs=[_row_spec(16), _stack_spec(2, 16),
                  _full_spec((16, 64)), _full_spec((16, 64)), _full_spec((1, 64))],
        out_specs=[_stack_spec(4, 16), _row_spec(16)],
        out_shape=[jax.ShapeDtypeStruct((4, N, 16), jnp.float32),
                   jax.ShapeDtypeStruct((N, 16), jnp.float32)],
    )(inp16, aggI2, Ws1p, Wn1p, b1)


def _tc_block(fquad, aggF4, aggZ2, aggI, inp16,
              Ws64, Ws32, Ws16, Wn64, Wn32, Wn16, b, z):
    def body(f_r, af_r, az_r, ai_r, inp_r,
             Ws64_r, Ws32_r, Ws16_r, Wn64_r, Wn32_r, Wn16_r, b_r, z_r,
             f_o):
        f = f_r[...]
        feats = jnp.concatenate([f[0], f[1], f[2], f[3]], axis=1)
        af = af_r[...]
        aggF = jnp.concatenate([af[0], af[1], af[2], af[3]], axis=1)
        az = az_r[...]
        aggZ = jnp.concatenate([az[0], az[1]], axis=1)
        pre = (_dot(feats, Ws64_r[...]) + _dot(z_r[...], Ws32_r[...])
               + _dot(inp_r[...], Ws16_r[...])
               + _dot(aggF, Wn64_r[...]) + _dot(aggZ, Wn32_r[...])
               + _dot(ai_r[...], Wn16_r[...]) + b_r[...])
        feats = jnp.maximum(pre, 0.0)
        for q in range(4):
            f_o[q] = feats[:, 16 * q:16 * (q + 1)]

    return pl.pallas_call(
        body,
        grid=(GRID,),
        in_specs=[_stack_spec(4, 16), _stack_spec(4, 16), _stack_spec(2, 16),
                  _row_spec(16), _row_spec(16),
                  _full_spec((64, 64)), _full_spec((32, 64)), _full_spec((16, 64)),
                  _full_spec((64, 64)), _full_spec((32, 64)), _full_spec((16, 64)),
                  _full_spec((1, 64)), _row_spec(32)],
        out_specs=_stack_spec(4, 16),
        out_shape=jax.ShapeDtypeStruct((4, N, 16), jnp.float32),
    )(fquad, aggF4, aggZ2, aggI, inp16, Ws64, Ws32, Ws16, Wn64, Wn32, Wn16, b, z)


def _tc_last(fquad, aggF4, Wsl16, Wnl16, bl16):
    def body(f_r, af_r, Wsl_r, Wnl_r, bl_r, o):
        f = f_r[...]
        feats = jnp.concatenate([f[0], f[1], f[2], f[3]], axis=1)
        af = af_r[...]
        aggF = jnp.concatenate([af[0], af[1], af[2], af[3]], axis=1)
        o[...] = _dot(feats, Wsl_r[...]) + _dot(aggF, Wnl_r[...]) + bl_r[...]

    return pl.pallas_call(
        body,
        grid=(GRID,),
        in_specs=[_stack_spec(4, 16), _stack_spec(4, 16),
                  _full_spec((64, 16)), _full_spec((64, 16)), _full_spec((1, 16))],
        out_specs=_row_spec(16),
        out_shape=jax.ShapeDtypeStruct((N, 16), jnp.float32),
    )(fquad, aggF4, Wsl16, Wnl16, bl16)


def kernel(vertices, edges, vector_potential, z, params):
    f32 = jnp.float32
    src = edges[:, 0]
    dst = edges[:, 1]
    pad = E_PAD - E
    srcp = jnp.concatenate([src, jnp.zeros((pad,), jnp.int32)])
    dstp = jnp.concatenate([dst, jnp.full((pad,), N, jnp.int32)])
    src4 = jnp.concatenate([srcp, srcp + N, srcp + 2 * N, srcp + 3 * N])

    inp16 = jnp.pad(jnp.concatenate([vertices, vector_potential], axis=-1),
                    ((0, 0), (0, 16 - 2 * 3)))
    zc16 = jnp.zeros((RPT, 16), f32)

    Ws1, Wn1, b1 = params["first"]
    Ws1p = jnp.pad(Ws1, ((0, 16 - DIN), (0, 0)))
    Wn1p = jnp.pad(Wn1, ((0, 16 - DIN), (0, 0)))
    b1r = b1.reshape(1, H)

    blocks = []
    for (Ws, Wn, b) in params["blocks"]:
        blocks.append((Ws[:H], Ws[H:H + DZ], jnp.pad(Ws[H + DZ:], ((0, 16 - DIN), (0, 0))),
                       Wn[:H], Wn[H:H + DZ], jnp.pad(Wn[H + DZ:], ((0, 16 - DIN), (0, 0))),
                       b.reshape(1, H)))

    Wsl, Wnl, bl = params["last"]
    Wsl16 = jnp.pad(Wsl, ((0, 0), (0, 16 - DOUT)))
    Wnl16 = jnp.pad(Wnl, ((0, 0), (0, 16 - DOUT)))
    bl16 = jnp.pad(bl, (0, 16 - DOUT)).reshape(1, 16)

    aggI2 = _segsum_rows16(inp16, srcp, dstp, zc16)
    zq01 = jnp.concatenate([z[0, :, :16], z[0, :, 16:],
                            z[1, :, :16], z[1, :, 16:]], axis=0)
    zq23 = jnp.concatenate([z[2, :, :16], z[2, :, 16:],
                            z[3, :, :16], z[3, :, 16:]], axis=0)
    aggZ01 = _segsum_cols(zq01, src4, dstp, zc16)
    aggZ23 = _segsum_cols(zq23, src4, dstp, zc16)
    aggZ = [aggZ01[0:2], aggZ01[2:4], aggZ23[0:2], aggZ23[2:4]]

    fquad, aggI = _tc_first(inp16, aggI2, Ws1p, Wn1p, b1r)

    for i in range(4):
        aggF4 = _segsum_cols(fquad.reshape(4 * N, 16), src4, dstp, zc16)
        fquad = _tc_block(fquad, aggF4, aggZ[i], aggI,
                          inp16, *blocks[i], z[i])

    aggF4 = _segsum_cols(fquad.reshape(4 * N, 16), src4, dstp, zc16)
    out16 = _tc_last(fquad, aggF4, Wsl16, Wnl16, bl16)
    return out16[:, :DOUT]
